# Initial kernel scaffold; baseline (speedup 1.0000x reference)
import jax
import jax.numpy as jnp
from jax.experimental import pallas as pl


def kernel(z_global, z_local, z_global_local_features, z_local_local_features, grid_global, grid_local):
    raise NotImplementedError("write your pallas kernel here")


# speedup vs baseline: 2.2958x; 2.2958x over previous
import functools

import jax
import jax.numpy as jnp
from jax.experimental import pallas as pl
from jax.experimental.pallas import tpu as pltpu

LAMBDA = 25.0
MU = 25.0
NU = 1.0
ALPHA = 0.25
EPS = 1e-4
NUM_MATCHES = (20, 4)

_BB = 16


def _global_body(za_ref, zb_ref, out_ref):
    za = za_ref[...]
    zb = zb_ref[...]
    n, d = za.shape
    diff = za - zb
    inv_sum = jnp.sum(diff * diff)

    def stats(x):
        s1 = jnp.sum(x, axis=0, keepdims=True)
        s2 = jnp.sum(x * x, axis=0, keepdims=True)
        mu = s1 / n
        dvec = s2 - n * mu * mu
        varc = dvec / (n - 1)
        std = jnp.sqrt(varc + EPS)
        var_loss = jnp.mean(jnp.maximum(1.0 - std, 0.0))
        xc = x - mu
        g = jax.lax.dot_general(xc, xc, (((1,), (1,)), ((), ())),
                                preferred_element_type=jnp.float32)
        gf2 = jnp.sum(g * g)
        cov_loss = (gf2 - jnp.sum(dvec * dvec)) / ((n - 1.0) ** 2) / d
        return var_loss, cov_loss

    va, ca = stats(za)
    vb, cb = stats(zb)
    gl = (LAMBDA * (inv_sum / (n * d))
          + MU * 0.5 * (va + vb)
          + NU * (ca + cb))
    out_ref[...] = jnp.full((8, 128), gl, jnp.float32)


def _rank_mask_sum(vals, gather, k):
    bb, L = vals.shape
    vi = vals[:, :, None]
    vj = vals[:, None, :]
    ii = jax.lax.broadcasted_iota(jnp.int32, (bb, L, L), 1)
    jj = jax.lax.broadcasted_iota(jnp.int32, (bb, L, L), 2)
    before = (vj < vi) | ((vj == vi) & (jj < ii))
    rank = jnp.sum(before.astype(jnp.int32), axis=-1)
    return jnp.sum(jnp.where(rank < k, gather, 0.0))


def _local_body(zg_ref, zl_ref, ggx_ref, ggy_ref, glx_ref, gly_ref, out_ref):
    zg = zg_ref[...]
    zl = zl_ref[...]
    ggx = ggx_ref[...]
    ggy = ggy_ref[...]
    glx = glx_ref[...]
    gly = gly_ref[...]
    bb, li, _ = zg.shape
    lj = zl.shape[1]

    featrows = []
    gridrows = []
    for j in range(lj):
        dz = zg - zl[:, j:j + 1, :]
        featrows.append(jnp.sum(dz * dz, axis=-1))
        dx = ggx - glx[:, j:j + 1]
        dy = ggy - gly[:, j:j + 1]
        gridrows.append(dx * dx + dy * dy)

    nn_feat_g = featrows[0]
    nn_grid_g = gridrows[0]
    featsel_g = featrows[0]
    for j in range(1, lj):
        nn_feat_g = jnp.minimum(nn_feat_g, featrows[j])
        upd = gridrows[j] < nn_grid_g
        nn_grid_g = jnp.where(upd, gridrows[j], nn_grid_g)
        featsel_g = jnp.where(upd, featrows[j], featsel_g)

    featT = jnp.stack(featrows, axis=1)
    gridT = jnp.stack(gridrows, axis=1)
    nn_feat_l = jnp.min(featT, axis=-1)
    nn_grid_l = jnp.min(gridT, axis=-1)
    gmin = nn_grid_l[:, :, None]
    iota_i = jax.lax.broadcasted_iota(jnp.int32, (bb, lj, li), 2)
    idx = jnp.min(jnp.where(gridT == gmin, iota_i, li), axis=-1, keepdims=True)
    featsel_l = jnp.sum(jnp.where(iota_i == idx, featT, 0.0), axis=-1)

    s_gf = _rank_mask_sum(nn_feat_g, nn_feat_g, NUM_MATCHES[0])
    s_gg = _rank_mask_sum(nn_grid_g, featsel_g, NUM_MATCHES[0])
    s_lf = _rank_mask_sum(nn_feat_l, nn_feat_l, NUM_MATCHES[1])
    s_lg = _rank_mask_sum(nn_grid_l, featsel_l, NUM_MATCHES[1])

    lane = jax.lax.broadcasted_iota(jnp.int32, (1, 1, 128), 2)
    row = (jnp.where(lane == 0, s_gf, 0.0)
           + jnp.where(lane == 1, s_gg, 0.0)
           + jnp.where(lane == 2, s_lf, 0.0)
           + jnp.where(lane == 3, s_lg, 0.0))
    out_ref[...] = row


@jax.jit
def kernel(z_global, z_local, z_global_local_features, z_local_local_features,
           grid_global, grid_local):
    B = z_global_local_features.shape[0]
    D = z_global_local_features.shape[-1]
    zg = z_global_local_features.reshape(B, -1, D)
    zl = z_local_local_features.reshape(B, -1, D)
    gg = grid_global.reshape(B, -1, 2)
    gl = grid_local.reshape(B, -1, 2)
    ggx, ggy = gg[..., 0], gg[..., 1]
    glx, gly = gl[..., 0], gl[..., 1]
    li, lj = zg.shape[1], zl.shape[1]

    global_out = pl.pallas_call(
        _global_body,
        out_shape=jax.ShapeDtypeStruct((8, 128), jnp.float32),
    )(z_global, z_local)

    nb = B // _BB
    local_out = pl.pallas_call(
        _local_body,
        grid=(nb,),
        in_specs=[
            pl.BlockSpec((_BB, li, D), lambda i: (i, 0, 0)),
            pl.BlockSpec((_BB, lj, D), lambda i: (i, 0, 0)),
            pl.BlockSpec((_BB, li), lambda i: (i, 0)),
            pl.BlockSpec((_BB, li), lambda i: (i, 0)),
            pl.BlockSpec((_BB, lj), lambda i: (i, 0)),
            pl.BlockSpec((_BB, lj), lambda i: (i, 0)),
        ],
        out_specs=pl.BlockSpec((1, 1, 128), lambda i: (i, 0, 0)),
        out_shape=jax.ShapeDtypeStruct((nb, 1, 128), jnp.float32),
    )(zg, zl, ggx, ggy, glx, gly)

    sums = jnp.sum(local_out.reshape(nb, 128), axis=0)
    cg = B * NUM_MATCHES[0] * D
    cl = B * NUM_MATCHES[1] * D
    inv_loss = 0.5 * (sums[0] / cg + sums[2] / cl + sums[1] / cg + sums[3] / cl)
    local_loss = LAMBDA * inv_loss
    global_loss = global_out[0, 0]
    return ALPHA * global_loss + (1.0 - ALPHA) * local_loss


# speedup vs baseline: 3.8605x; 1.6815x over previous
import functools

import jax
import jax.numpy as jnp
from jax.experimental import pallas as pl
from jax.experimental.pallas import tpu as pltpu

LAMBDA = 25.0
MU = 25.0
NU = 1.0
ALPHA = 0.25
EPS = 1e-4
NUM_MATCHES = (20, 4)

_BB = 16
_LI = 49
_LJ = 9
_D = 512


def _global_body(za_ref, zb_ref, out_ref):
    za = za_ref[...]
    zb = zb_ref[...]
    n, d = za.shape
    diff = za - zb
    inv_sum = jnp.sum(diff * diff)

    def stats(x):
        s1 = jnp.sum(x, axis=0, keepdims=True)
        s2 = jnp.sum(x * x, axis=0, keepdims=True)
        mu = s1 / n
        dvec = s2 - n * mu * mu
        varc = dvec / (n - 1)
        std = jnp.sqrt(varc + EPS)
        var_loss = jnp.mean(jnp.maximum(1.0 - std, 0.0))
        xc = x - mu
        g = jax.lax.dot_general(xc, xc, (((1,), (1,)), ((), ())),
                                preferred_element_type=jnp.float32)
        gf2 = jnp.sum(g * g)
        cov_loss = (gf2 - jnp.sum(dvec * dvec)) / ((n - 1.0) ** 2) / d
        return var_loss, cov_loss

    va, ca = stats(za)
    vb, cb = stats(zb)
    gl = (LAMBDA * (inv_sum / (n * d))
          + MU * 0.5 * (va + vb)
          + NU * (ca + cb))
    out_ref[...] = jnp.full((8, 128), gl, jnp.float32)


def _rank_mask_sum(vals, gather, k):
    bb, L = vals.shape
    vi = vals[:, :, None]
    vj = vals[:, None, :]
    ii = jax.lax.broadcasted_iota(jnp.int32, (bb, L, L), 1)
    jj = jax.lax.broadcasted_iota(jnp.int32, (bb, L, L), 2)
    before = (vj < vi) | ((vj == vi) & (jj < ii))
    rank = jnp.sum(before.astype(jnp.int32), axis=-1)
    return jnp.sum(jnp.where(rank < k, gather, 0.0))


def _local_body(zgf_ref, zlf_ref, ggx_ref, ggy_ref, glx_ref, gly_ref, out_ref):
    zgf = zgf_ref[...]
    zlf = zlf_ref[...]
    ggx = ggx_ref[...]
    ggy = ggy_ref[...]
    glx = glx_ref[...]
    gly = gly_ref[...]
    bb = _BB
    nr = bb * _LJ
    nc = bb * _LI

    dims = (((1,), (1,)), ((), ()))
    cross = jax.lax.dot_general(zlf, zgf, dims,
                                preferred_element_type=jnp.float32)
    ng = jax.lax.dot_general(jnp.ones((nr, _D), jnp.float32), zgf * zgf, dims,
                             preferred_element_type=jnp.float32)
    nl = jnp.sum(zlf * zlf, axis=1, keepdims=True)
    d2t = ng + nl - 2.0 * cross

    F = jnp.stack([d2t[_LJ * b:_LJ * (b + 1), _LI * b:_LI * (b + 1)]
                   for b in range(bb)], axis=0)

    gxj = glx[:, :, None]
    gyj = gly[:, :, None]
    gxi = ggx[:, None, :]
    gyi = ggy[:, None, :]
    Gd = (gxi - gxj) ** 2 + (gyi - gyj) ** 2

    nn_feat_g = jnp.min(F, axis=1)
    nn_grid_g = jnp.min(Gd, axis=1)
    iota_j = jax.lax.broadcasted_iota(jnp.int32, (bb, _LJ, _LI), 1)
    idxj = jnp.min(jnp.where(Gd == nn_grid_g[:, None, :], iota_j, _LJ),
                   axis=1, keepdims=True)
    featsel_g = jnp.sum(jnp.where(iota_j == idxj, F, 0.0), axis=1)

    nn_feat_l = jnp.min(F, axis=2)
    nn_grid_l = jnp.min(Gd, axis=2)
    iota_i = jax.lax.broadcasted_iota(jnp.int32, (bb, _LJ, _LI), 2)
    idxi = jnp.min(jnp.where(Gd == nn_grid_l[:, :, None], iota_i, _LI),
                   axis=2, keepdims=True)
    featsel_l = jnp.sum(jnp.where(iota_i == idxi, F, 0.0), axis=2)

    s_gf = _rank_mask_sum(nn_feat_g, nn_feat_g, NUM_MATCHES[0])
    s_gg = _rank_mask_sum(nn_grid_g, featsel_g, NUM_MATCHES[0])
    s_lf = _rank_mask_sum(nn_feat_l, nn_feat_l, NUM_MATCHES[1])
    s_lg = _rank_mask_sum(nn_grid_l, featsel_l, NUM_MATCHES[1])

    lane = jax.lax.broadcasted_iota(jnp.int32, (1, 1, 128), 2)
    row = (jnp.where(lane == 0, s_gf, 0.0)
           + jnp.where(lane == 1, s_gg, 0.0)
           + jnp.where(lane == 2, s_lf, 0.0)
           + jnp.where(lane == 3, s_lg, 0.0))
    out_ref[...] = row


@jax.jit
def kernel(z_global, z_local, z_global_local_features, z_local_local_features,
           grid_global, grid_local):
    B = z_global_local_features.shape[0]
    D = z_global_local_features.shape[-1]
    zgf = z_global_local_features.reshape(B * _LI, D)
    zlf = z_local_local_features.reshape(B * _LJ, D)
    gg = grid_global.reshape(B, -1, 2)
    gl = grid_local.reshape(B, -1, 2)
    ggx, ggy = gg[..., 0], gg[..., 1]
    glx, gly = gl[..., 0], gl[..., 1]

    global_out = pl.pallas_call(
        _global_body,
        out_shape=jax.ShapeDtypeStruct((8, 128), jnp.float32),
    )(z_global, z_local)

    nb = B // _BB
    local_out = pl.pallas_call(
        _local_body,
        grid=(nb,),
        in_specs=[
            pl.BlockSpec((_BB * _LI, D), lambda i: (i, 0)),
            pl.BlockSpec((_BB * _LJ, D), lambda i: (i, 0)),
            pl.BlockSpec((_BB, _LI), lambda i: (i, 0)),
            pl.BlockSpec((_BB, _LI), lambda i: (i, 0)),
            pl.BlockSpec((_BB, _LJ), lambda i: (i, 0)),
            pl.BlockSpec((_BB, _LJ), lambda i: (i, 0)),
        ],
        out_specs=pl.BlockSpec((1, 1, 128), lambda i: (i, 0, 0)),
        out_shape=jax.ShapeDtypeStruct((nb, 1, 128), jnp.float32),
    )(zgf, zlf, ggx, ggy, glx, gly)

    sums = jnp.sum(local_out.reshape(nb, 128), axis=0)
    cg = B * NUM_MATCHES[0] * D
    cl = B * NUM_MATCHES[1] * D
    inv_loss = 0.5 * (sums[0] / cg + sums[2] / cl + sums[1] / cg + sums[3] / cl)
    local_loss = LAMBDA * inv_loss
    global_loss = global_out[0, 0]
    return ALPHA * global_loss + (1.0 - ALPHA) * local_loss
